# baseline (speedup 1.0000x reference)
import jax
import jax.numpy as jnp
from jax import lax
from jax.experimental import pallas as pl
from jax.experimental.pallas import tpu as pltpu
from jax.experimental.pallas import tpu_sc as plsc

B, L = 4096, 200
D = 64
N = B * L
NC, NS = 2, 16
NW = NC * NS
BT = B // NW
NL = 2
ITERS = L // NL
ROWS_IDX = N // BT


def _prep_kernel(type_ref, hyp_ref, pos_ref, typ_ids_ref, hyp_ids_ref,
                 pos_ids_ref, comb_ref, cidx_ref):
  comb_ref[...] = (
      type_ref[...][:, None, None, :]
      + hyp_ref[...][None, :, None, :]
      + pos_ref[...][None, None, :, :]
  )
  cidx_ref[...] = (
      typ_ids_ref[...] * 1024 + hyp_ids_ref[...] * 512 + pos_ids_ref[...]
  )


def _sc_body(tok_hbm, cidx_hbm, emb_hbm, comb_hbm, out_hbm,
             tok_v, cidx_v, emb_rows, comb_rows, brick,
             semg0, semg1, semo0, semo1):
  w = lax.axis_index("s") * NC + lax.axis_index("c")
  semg = (semg0, semg1)
  semo = (semo0, semo1)

  lane = lax.iota(jnp.int32, 16)
  dts = [(g * 16 + lane) >> 3 for g in range(D // 16)]
  dss = [(g * 16 + lane) & 7 for g in range(D // 16)]

  def issue_gathers(i, b):
    pltpu.sync_copy(tok_hbm.at[pl.ds(i * NL, NL), w], tok_v.at[b])
    pltpu.sync_copy(cidx_hbm.at[pl.ds(i * NL, NL), w], cidx_v.at[b])
    for k in range(NL):
      pltpu.async_copy(emb_hbm.at[tok_v.at[b, k]],
                       emb_rows.at[b, k], semg[b])
      pltpu.async_copy(comb_hbm.at[cidx_v.at[b, k]],
                       comb_rows.at[b, k], semg[b])

  def wait_gathers(i, b):
    for k in range(NL):
      pltpu.make_async_copy(emb_hbm.at[tok_v.at[b, k]],
                            emb_rows.at[b, k], semg[b]).wait()
      pltpu.make_async_copy(comb_hbm.at[cidx_v.at[b, k]],
                            comb_rows.at[b, k], semg[b]).wait()

  def store(i, b, k):
    return pltpu.make_async_copy(brick.at[b, k],
                                 out_hbm.at[i * NL + k, :, w], semo[b])

  issue_gathers(0, 0)

  def body(g2, carry):
    for b in range(2):
      i = g2 * 2 + b

      @pl.when(i < ITERS - 1)
      def _():
        issue_gathers(i + 1, 1 - b)

      wait_gathers(i, b)

      @pl.when(i >= 2)
      def _():
        for k in range(NL):
          store(i - 2, b, k).wait()

      for k in range(NL):
        def add_tok(c, carry2):
          blv = jnp.full((16,), c, jnp.int32)
          for gg in range(D // 16):
            sl = pl.ds(gg * 16, 16)
            v = emb_rows[b, k, c, sl] + comb_rows[b, k, c, sl]
            plsc.store_scatter(brick.at[b, k], [dts[gg], dss[gg], blv], v)
          return carry2

        lax.fori_loop(0, BT, add_tok, 0, unroll=2)
      for k in range(NL):
        store(i, b, k).start()
    return carry

  lax.fori_loop(0, ITERS // 2, body, 0)
  for k in range(NL):
    store(ITERS - 2, 0, k).wait()
    store(ITERS - 1, 1, k).wait()


def kernel(input_ids, token_type_ids, position_ids, if_hyp_ids,
           embed_table, type_table, pos_table, hyp_table):
  tok = input_ids.astype(jnp.int32).T.reshape(L, NW, BT)
  typ = token_type_ids.astype(jnp.int32).T.reshape(ROWS_IDX, BT)
  pos = position_ids.astype(jnp.int32).T.reshape(ROWS_IDX, BT)
  hyp = if_hyp_ids.astype(jnp.int32).T.reshape(ROWS_IDX, BT)

  comb4, cidx = pl.pallas_call(
      _prep_kernel,
      out_shape=(
          jax.ShapeDtypeStruct((2, 2, 512, D), jnp.float32),
          jax.ShapeDtypeStruct((ROWS_IDX, BT), jnp.int32),
      ),
  )(type_table, hyp_table, pos_table, typ, hyp, pos)
  comb = comb4.reshape(2 * 2 * 512, D)

  mesh = plsc.VectorSubcoreMesh(core_axis_name="c", subcore_axis_name="s")
  sc = pl.kernel(
      _sc_body,
      out_type=jax.ShapeDtypeStruct((L, D // 8, NW, 8, BT), jnp.float32),
      mesh=mesh,
      compiler_params=pltpu.CompilerParams(use_tc_tiling_on_sc=False,
                                           needs_layout_passes=False),
      scratch_types=[
          pltpu.VMEM((2, NL, BT), jnp.int32),
          pltpu.VMEM((2, NL, BT), jnp.int32),
          pltpu.VMEM((2, NL, BT, D), jnp.float32),
          pltpu.VMEM((2, NL, BT, D), jnp.float32),
          pltpu.VMEM((2, NL, D // 8, 8, BT), jnp.float32),
          pltpu.SemaphoreType.DMA,
          pltpu.SemaphoreType.DMA,
          pltpu.SemaphoreType.DMA,
          pltpu.SemaphoreType.DMA,
      ],
  )
  out_p = sc(tok, cidx.reshape(L, NW, BT), embed_table, comb)
  return out_p.transpose(2, 4, 0, 1, 3).reshape(B, L, D)


# speedup vs baseline: 1.7199x; 1.7199x over previous
import jax
import jax.numpy as jnp
from jax import lax
from jax.experimental import pallas as pl
from jax.experimental.pallas import tpu as pltpu
from jax.experimental.pallas import tpu_sc as plsc

B, L = 4096, 200
D = 64
N = B * L
NC, NS = 2, 16
NW = NC * NS
BT = B // NW
NL = 2
ITERS = L // NL
ROWS_IDX = N // BT


def _prep_kernel(type_ref, hyp_ref, pos_ref, typ_ids_ref, hyp_ids_ref,
                 pos_ids_ref, comb_ref, cidx_ref):
  comb_ref[...] = (
      type_ref[...][:, None, None, :]
      + hyp_ref[...][None, :, None, :]
      + pos_ref[...][None, None, :, :]
  )
  cidx_ref[...] = (
      typ_ids_ref[...] * 1024 + hyp_ids_ref[...] * 512 + pos_ids_ref[...]
  )


def _sc_body(tok_hbm, cidx_hbm, emb_hbm, comb_hbm, out_hbm,
             tok_v, cidx_v, emb_rows, comb_rows, brick,
             semg0, semg1, semo0, semo1):
  w = lax.axis_index("s") * NC + lax.axis_index("c")
  semg = (semg0, semg1)
  semo = (semo0, semo1)

  lane = lax.iota(jnp.int32, 16)
  dts = [(g * 16 + lane) >> 3 for g in range(D // 16)]
  dss = [(g * 16 + lane) & 7 for g in range(D // 16)]

  def issue_gathers(i, b):
    pltpu.sync_copy(tok_hbm.at[pl.ds(i * NL, NL), w], tok_v.at[b])
    pltpu.sync_copy(cidx_hbm.at[pl.ds(i * NL, NL), w], cidx_v.at[b])
    for k in range(NL):
      pltpu.async_copy(emb_hbm.at[tok_v.at[b, k]],
                       emb_rows.at[b, k], semg[b])
      pltpu.async_copy(comb_hbm.at[cidx_v.at[b, k]],
                       comb_rows.at[b, k], semg[b])

  def wait_gathers(i, b):
    for k in range(NL):
      pltpu.make_async_copy(emb_hbm.at[tok_v.at[b, k]],
                            emb_rows.at[b, k], semg[b]).wait()
      pltpu.make_async_copy(comb_hbm.at[cidx_v.at[b, k]],
                            comb_rows.at[b, k], semg[b]).wait()

  def store(i, b, k):
    return pltpu.make_async_copy(brick.at[b, k, :, :, pl.ds(0, BT)],
                                 out_hbm.at[i * NL + k, :, w], semo[b])

  issue_gathers(0, 0)

  def body(g2, carry):
    for b in range(2):
      i = g2 * 2 + b

      @pl.when(i < ITERS - 1)
      def _():
        issue_gathers(i + 1, 1 - b)

      wait_gathers(i, b)

      @pl.when(i >= 2)
      def _():
        for k in range(NL):
          store(i - 2, b, k).wait()

      for k in range(NL):
        def add_tok(c, carry2):
          blv = jnp.full((16,), c, jnp.int32)
          for gg in range(D // 16):
            sl = pl.ds(gg * 16, 16)
            v = emb_rows[b, k, c, sl] + comb_rows[b, k, c, sl]
            plsc.store_scatter(brick.at[b, k], [dts[gg], dss[gg], blv], v)
          return carry2

        lax.fori_loop(0, BT, add_tok, 0, unroll=2)
      for k in range(NL):
        store(i, b, k).start()
    return carry

  lax.fori_loop(0, ITERS // 2, body, 0)
  for k in range(NL):
    store(ITERS - 2, 0, k).wait()
    store(ITERS - 1, 1, k).wait()


def kernel(input_ids, token_type_ids, position_ids, if_hyp_ids,
           embed_table, type_table, pos_table, hyp_table):
  tok = input_ids.astype(jnp.int32).T.reshape(L, NW, BT)
  typ = token_type_ids.astype(jnp.int32).T.reshape(ROWS_IDX, BT)
  pos = position_ids.astype(jnp.int32).T.reshape(ROWS_IDX, BT)
  hyp = if_hyp_ids.astype(jnp.int32).T.reshape(ROWS_IDX, BT)

  comb4, cidx = pl.pallas_call(
      _prep_kernel,
      out_shape=(
          jax.ShapeDtypeStruct((2, 2, 512, D), jnp.float32),
          jax.ShapeDtypeStruct((ROWS_IDX, BT), jnp.int32),
      ),
  )(type_table, hyp_table, pos_table, typ, hyp, pos)
  comb = comb4.reshape(2 * 2 * 512, D)

  mesh = plsc.VectorSubcoreMesh(core_axis_name="c", subcore_axis_name="s")
  sc = pl.kernel(
      _sc_body,
      out_type=jax.ShapeDtypeStruct((L, D // 8, NW, 8, BT), jnp.float32),
      mesh=mesh,
      compiler_params=pltpu.CompilerParams(use_tc_tiling_on_sc=False,
                                           needs_layout_passes=False),
      scratch_types=[
          pltpu.VMEM((2, NL, BT), jnp.int32),
          pltpu.VMEM((2, NL, BT), jnp.int32),
          pltpu.VMEM((2, NL, BT, D), jnp.float32),
          pltpu.VMEM((2, NL, BT, D), jnp.float32),
          pltpu.VMEM((2, NL, D // 8, 8, BT + 1), jnp.float32),
          pltpu.SemaphoreType.DMA,
          pltpu.SemaphoreType.DMA,
          pltpu.SemaphoreType.DMA,
          pltpu.SemaphoreType.DMA,
      ],
  )
  out_p = sc(tok, cidx.reshape(L, NW, BT), embed_table, comb)
  return out_p.transpose(2, 4, 0, 1, 3).reshape(B, L, D)
